# baseline (speedup 1.0000x reference)
import functools

import jax
import jax.numpy as jnp
from jax import lax
from jax.experimental import pallas as pl
from jax.experimental.pallas import tpu as pltpu
from jax.experimental.pallas import tpu_sc as plsc

V = 100000
D = 128
B = 4096
MOM = 0.5
NC, NS, L = 2, 16, 16
NW = NC * NS
R = V // NW
CH = 128
CAP = B + 2 * L
TRASH = CAP - 1
DB = D // L

_mesh = plsc.VectorSubcoreMesh(core_axis_name="c", subcore_axis_name="s")


@functools.partial(
    pl.kernel,
    out_type=(),
    mesh=_mesh,
    compiler_params=pltpu.CompilerParams(needs_layout_passes=False),
    scratch_types=[
        pltpu.VMEM((B,), jnp.int32),
        pltpu.VMEM((CAP,), jnp.int32),
        pltpu.VMEM((CAP,), jnp.int32),
        pltpu.VMEM((CAP,), jnp.int32),
        pltpu.VMEM((B // CH, CH), jnp.int32),
        pltpu.VMEM((CH, D), jnp.float32),
        pltpu.VMEM((CH, D), jnp.float32),
        pltpu.SemaphoreType.DMA,
        pltpu.SemaphoreType.DMA,
    ],
)
def _sc_update(x_hbm, y_hbm, mem_hbm, out_ref,
               y_v, pos_v, idx_v, last_v, idx2, mrow, xrow, semA, semB):
    wid = lax.axis_index("s") * NC + lax.axis_index("c")
    lo = wid * R
    hi = lo + R
    lanes = lax.iota(jnp.int32, L)

    pltpu.sync_copy(y_hbm, y_v)

    @pl.loop(0, B // L, init_carry=jnp.int32(0))
    def compact(i, cnt):
        yv = y_v[pl.ds(i * L, L)]
        m = (yv >= jnp.full((L,), lo, jnp.int32)) & (yv < jnp.full((L,), hi, jnp.int32))
        mi = jnp.where(m, jnp.full((L,), 1, jnp.int32), jnp.full((L,), 0, jnp.int32))
        slots = jnp.where(m, plsc.cumsum(mi) + jnp.full((L,), cnt - 1, jnp.int32),
                          jnp.full((L,), TRASH, jnp.int32))
        plsc.store_scatter(pos_v, [slots], i * L + lanes)
        plsc.store_scatter(idx_v, [slots], yv)
        return cnt + jnp.sum(mi)

    cnt = compact

    @pl.when(cnt > 0)
    def _():
        nblk = (cnt + L - 1) // L
        nch = (cnt + CH - 1) // CH
        pend = nch * CH

        last_idx = idx_v[pl.ds(cnt - 1, L)][0]
        last_pos = pos_v[pl.ds(cnt - 1, L)][0]

        @pl.loop(0, pend // L)
        def fill(b):
            base = b * L
            slot = base + lanes
            live = slot < cnt
            cur_i = idx_v[pl.ds(base, L)]
            cur_p = pos_v[pl.ds(base, L)]
            idx_v[pl.ds(base, L)] = jnp.where(live, cur_i, last_idx)
            pos_v[pl.ds(base, L)] = jnp.where(live, cur_p, last_pos)
            last_v[pl.ds(base, L)] = slot

        @pl.loop(0, pend)
        def dedup(k):
            vk = idx_v[pl.ds(k, L)][0]

            @pl.loop(0, pend // L)
            def blk(b):
                base = b * L
                eq = idx_v[pl.ds(base, L)] == vk
                cur = last_v[pl.ds(base, L)]
                last_v[pl.ds(base, L)] = jnp.where(eq, k, cur)

        @pl.loop(0, pend // L)
        def rewrite(b):
            base = b * L
            w = last_v[pl.ds(base, L)]
            pos_v[pl.ds(base, L)] = plsc.load_gather(pos_v, [w])

        @pl.loop(0, nch)
        def chunk(c):
            off = c * CH
            for b in range(CH // L):
                idx2[c, pl.ds(b * L, L)] = idx_v[pl.ds(off + b * L, L)]
            gm = pltpu.async_copy(mem_hbm.at[idx2.at[c]], mrow, semA)
            gx = pltpu.async_copy(x_hbm.at[pos_v.at[pl.ds(off, CH)]], xrow, semB)
            gm.wait()
            gx.wait()

            @pl.loop(0, CH)
            def row(r):
                acc = jnp.zeros((L,), jnp.float32)
                for dblk in range(DB):
                    s = pl.ds(dblk * L, L)
                    v = mrow[r, s] * MOM + xrow[r, s] * (1.0 - MOM)
                    mrow[r, s] = v
                    acc = acc + v * v
                ss = jnp.full((L,), jnp.sum(acc), jnp.float32)
                bits = plsc.bitcast(ss, jnp.int32)
                guess = plsc.bitcast(
                    jnp.full((L,), 0x5F3759DF, jnp.int32) - (bits >> 1),
                    jnp.float32)
                for _ in range(3):
                    guess = guess * (1.5 - 0.5 * ss * guess * guess)
                for dblk in range(DB):
                    s = pl.ds(dblk * L, L)
                    mrow[r, s] = mrow[r, s] * guess

            sc = pltpu.async_copy(mrow, out_ref.at[idx2.at[c]], semA)
            sc.wait()


def kernel(x, x2, y, memory):
    mem_ref = jax.new_ref(memory)
    _sc_update(x, y, memory, mem_ref)
    return (x, x2, mem_ref[...])
